# baseline (device time: 18259 ns/iter reference)
import jax
import jax.numpy as jnp
from jax import lax
from jax.experimental import pallas as pl
from jax.experimental.pallas import tpu as pltpu

N_DEV = 16
N_TOK = 512
D_IN = 256
D_OUT = 512
N_EXP = 32
CHUNK = N_TOK // N_DEV
NBLK = 4
BLK = N_TOK // NBLK


def kernel(x, router_W, route_idx, expert_W):
    def body(x_ref, rw_ref, idx_ref, ew_ref, out_ref,
             p_ref, recv_buf, x2_ref, idx2_ref, send_sems, recv_sems):
        my = lax.axis_index("i")

        barrier_sem = pltpu.get_barrier_semaphore()
        for o in range(1, N_DEV):
            pl.semaphore_signal(
                barrier_sem, inc=1,
                device_id=(lax.rem(my + o, N_DEV),),
                device_id_type=pl.DeviceIdType.MESH,
            )

        shift = lax.rem(my + 1, N_DEV) * CHUNK
        x2_ref[0:N_TOK, :] = x_ref[:, :]
        x2_ref[N_TOK:2 * N_TOK, :] = x_ref[:, :]
        idx2_ref[0:N_TOK, :] = idx_ref[:, :]
        idx2_ref[N_TOK:2 * N_TOK, :] = idx_ref[:, :]
        xr = x2_ref[pl.ds(shift, N_TOK), :]
        idxr = idx2_ref[pl.ds(shift, N_TOK), :]

        scores = jnp.dot(xr, rw_ref[:, :], preferred_element_type=jnp.float32)
        s_max = jnp.max(scores, axis=-1, keepdims=True)
        pexp = jnp.exp(scores - s_max)
        probs = pexp / jnp.sum(pexp, axis=-1, keepdims=True)

        e_ids = lax.broadcasted_iota(jnp.int32, (N_TOK, N_EXP), 1)
        top_mask = (e_ids == idxr[:, 0:1]) | (e_ids == idxr[:, 1:2])
        gp = jnp.where(top_mask, probs, 0.0)
        gates = gp / jnp.sum(gp, axis=-1, keepdims=True)

        g0r = jnp.sum(jnp.where(e_ids == 2 * my, gates, 0.0),
                      axis=-1, keepdims=True)
        g1r = jnp.sum(jnp.where(e_ids == 2 * my + 1, gates, 0.0),
                      axis=-1, keepdims=True)

        pl.semaphore_wait(barrier_sem, N_DEV - 1)

        w0 = ew_ref[0]
        w1 = ew_ref[1]
        sends = []
        for blk in range(NBLK):
            r0 = blk * BLK
            xb = xr[r0:r0 + BLK, :]
            p_ref[pl.ds(r0, BLK), :] = (
                g0r[r0:r0 + BLK, :]
                * jnp.dot(xb, w0, preferred_element_type=jnp.float32)
                + g1r[r0:r0 + BLK, :]
                * jnp.dot(xb, w1, preferred_element_type=jnp.float32)
            )
            for k in range(blk * (N_DEV // NBLK), (blk + 1) * (N_DEV // NBLK)):
                if k == N_DEV - 1:
                    continue
                rdma = pltpu.make_async_remote_copy(
                    src_ref=p_ref.at[pl.ds(k * CHUNK, CHUNK), :],
                    dst_ref=recv_buf.at[k],
                    send_sem=send_sems.at[k],
                    recv_sem=recv_sems.at[k],
                    device_id=(lax.rem(my + 1 + k, N_DEV),),
                    device_id_type=pl.DeviceIdType.MESH,
                )
                rdma.start()
                sends.append(rdma)

        acc = p_ref[pl.ds((N_DEV - 1) * CHUNK, CHUNK), :]
        for k in range(N_DEV - 1):
            recv = pltpu.make_async_remote_copy(
                src_ref=recv_buf.at[k],
                dst_ref=recv_buf.at[k],
                send_sem=send_sems.at[k],
                recv_sem=recv_sems.at[k],
                device_id=(my,),
                device_id_type=pl.DeviceIdType.MESH,
            )
            recv.wait_recv()
            acc = acc + recv_buf[k]
        out_ref[:, :] = acc

        for rdma in sends:
            rdma.wait_send()

    return pl.pallas_call(
        body,
        out_shape=jax.ShapeDtypeStruct((CHUNK, D_OUT), jnp.float32),
        in_specs=[
            pl.BlockSpec(memory_space=pltpu.VMEM),
            pl.BlockSpec(memory_space=pltpu.VMEM),
            pl.BlockSpec(memory_space=pltpu.VMEM),
            pl.BlockSpec(memory_space=pltpu.VMEM),
        ],
        out_specs=pl.BlockSpec(memory_space=pltpu.VMEM),
        scratch_shapes=[
            pltpu.VMEM((N_TOK, D_OUT), jnp.float32),
            pltpu.VMEM((N_DEV - 1, CHUNK, D_OUT), jnp.float32),
            pltpu.VMEM((2 * N_TOK, D_IN), jnp.float32),
            pltpu.VMEM((2 * N_TOK, 2), jnp.int32),
            pltpu.SemaphoreType.DMA((N_DEV - 1,)),
            pltpu.SemaphoreType.DMA((N_DEV - 1,)),
        ],
        compiler_params=pltpu.CompilerParams(collective_id=0),
    )(x, router_W, route_idx, expert_W)


# device time: 15575 ns/iter; 1.1723x vs baseline; 1.1723x over previous
import jax
import jax.numpy as jnp
from jax import lax
from jax.experimental import pallas as pl
from jax.experimental.pallas import tpu as pltpu

N_DEV = 16
N_TOK = 512
D_IN = 256
D_OUT = 512
N_EXP = 32
CHUNK = N_TOK // N_DEV
NBLK = 2
BLK = N_TOK // NBLK


def kernel(x, router_W, route_idx, expert_W):
    def body(x_ref, rw_ref, idx_ref, ew_ref, out_ref,
             p_ref, recv_buf, x2_ref, idx2_ref, send_sems, recv_sems,
             credit_sems):
        my = lax.axis_index("i")

        for k in range(N_DEV - 1):
            pl.semaphore_signal(
                credit_sems.at[k], inc=1,
                device_id=(lax.rem(my + N_DEV - 1 - k, N_DEV),),
                device_id_type=pl.DeviceIdType.MESH,
            )
        barrier_sem = pltpu.get_barrier_semaphore()
        pl.semaphore_signal(
            barrier_sem, inc=1,
            device_id=(lax.rem(my + 1, N_DEV),),
            device_id_type=pl.DeviceIdType.MESH,
        )
        pl.semaphore_wait(barrier_sem, 1)

        shift = lax.rem(my + 1, N_DEV) * CHUNK
        x2_ref[0:N_TOK, :] = x_ref[:, :]
        x2_ref[N_TOK:2 * N_TOK, :] = x_ref[:, :]
        idx2_ref[0:N_TOK, :] = idx_ref[:, :]
        idx2_ref[N_TOK:2 * N_TOK, :] = idx_ref[:, :]
        xr = x2_ref[pl.ds(shift, N_TOK), :]
        idxr = idx2_ref[pl.ds(shift, N_TOK), :]

        scores = jnp.dot(xr, rw_ref[:, :], preferred_element_type=jnp.float32)
        s_max = jnp.max(scores, axis=-1, keepdims=True)
        pexp = jnp.exp(scores - s_max)
        probs = pexp / jnp.sum(pexp, axis=-1, keepdims=True)

        e_ids = lax.broadcasted_iota(jnp.int32, (N_TOK, N_EXP), 1)
        top_mask = (e_ids == idxr[:, 0:1]) | (e_ids == idxr[:, 1:2])
        gp = jnp.where(top_mask, probs, 0.0)
        gates = gp / jnp.sum(gp, axis=-1, keepdims=True)

        g0r = jnp.sum(jnp.where(e_ids == 2 * my, gates, 0.0),
                      axis=-1, keepdims=True)
        g1r = jnp.sum(jnp.where(e_ids == 2 * my + 1, gates, 0.0),
                      axis=-1, keepdims=True)

        w0 = ew_ref[0].astype(jnp.bfloat16)
        w1 = ew_ref[1].astype(jnp.bfloat16)
        xrb = xr.astype(jnp.bfloat16)
        sends = []
        for blk in range(NBLK):
            r0 = blk * BLK
            xb = xrb[r0:r0 + BLK, :]
            p_ref[pl.ds(r0, BLK), :] = (
                g0r[r0:r0 + BLK, :]
                * jnp.dot(xb, w0, preferred_element_type=jnp.float32)
                + g1r[r0:r0 + BLK, :]
                * jnp.dot(xb, w1, preferred_element_type=jnp.float32)
            ).astype(jnp.bfloat16)
            for k in range(blk * (N_DEV // NBLK), (blk + 1) * (N_DEV // NBLK)):
                if k == N_DEV - 1:
                    continue
                pl.semaphore_wait(credit_sems.at[k], 1)
                rdma = pltpu.make_async_remote_copy(
                    src_ref=p_ref.at[pl.ds(k * CHUNK, CHUNK), :],
                    dst_ref=recv_buf.at[k],
                    send_sem=send_sems.at[k],
                    recv_sem=recv_sems.at[k],
                    device_id=(lax.rem(my + 1 + k, N_DEV),),
                    device_id_type=pl.DeviceIdType.MESH,
                )
                rdma.start()
                sends.append(rdma)

        acc = p_ref[pl.ds((N_DEV - 1) * CHUNK, CHUNK), :].astype(
            jnp.float32)
        for k in range(N_DEV - 1):
            recv = pltpu.make_async_remote_copy(
                src_ref=recv_buf.at[k],
                dst_ref=recv_buf.at[k],
                send_sem=send_sems.at[k],
                recv_sem=recv_sems.at[k],
                device_id=(my,),
                device_id_type=pl.DeviceIdType.MESH,
            )
            recv.wait_recv()
            acc = acc + recv_buf[k].astype(jnp.float32)
        out_ref[:, :] = acc

        for rdma in sends:
            rdma.wait_send()

    return pl.pallas_call(
        body,
        out_shape=jax.ShapeDtypeStruct((CHUNK, D_OUT), jnp.float32),
        in_specs=[
            pl.BlockSpec(memory_space=pltpu.VMEM),
            pl.BlockSpec(memory_space=pltpu.VMEM),
            pl.BlockSpec(memory_space=pltpu.VMEM),
            pl.BlockSpec(memory_space=pltpu.VMEM),
        ],
        out_specs=pl.BlockSpec(memory_space=pltpu.VMEM),
        scratch_shapes=[
            pltpu.VMEM((N_TOK, D_OUT), jnp.bfloat16),
            pltpu.VMEM((N_DEV - 1, CHUNK, D_OUT), jnp.bfloat16),
            pltpu.VMEM((2 * N_TOK, D_IN), jnp.float32),
            pltpu.VMEM((2 * N_TOK, 2), jnp.int32),
            pltpu.SemaphoreType.DMA((N_DEV - 1,)),
            pltpu.SemaphoreType.DMA((N_DEV - 1,)),
            pltpu.SemaphoreType.REGULAR((N_DEV - 1,)),
        ],
        compiler_params=pltpu.CompilerParams(collective_id=0),
    )(x, router_W, route_idx, expert_W)
